# Initial kernel scaffold; baseline (speedup 1.0000x reference)
import functools

import jax
import jax.numpy as jnp
from jax import lax
from jax.experimental import pallas as pl
from jax.experimental.pallas import tpu as pltpu
from jax.experimental.pallas import tpu_sc as plsc

_L = 16


def _log_sigmoid_vec(x):
    e = jnp.exp(-jnp.abs(x))
    t = e / (e + 2.0)
    w = t * t
    poly = 1.0 + w * (1.0 / 3.0 + w * (1.0 / 5.0 + w * (1.0 / 7.0 + w * (1.0 / 9.0))))
    log1p_e = 2.0 * t * poly
    return jnp.minimum(x, 0.0) - log1p_e


def _make_sc_kernel(B, D, K, NC, NS):
    NW = NC * NS
    R = K + 1
    IPW = B // NW
    CHUNK = 16
    NCHUNK = IPW // CHUNK
    RPC = CHUNK * R
    GSPLIT = 112
    NG = RPC // GSPLIT
    GROUPS = RPC // _L
    DV = D // _L

    mesh = plsc.VectorSubcoreMesh(core_axis_name="c", subcore_axis_name="s")

    @functools.partial(
        pl.kernel,
        mesh=mesh,
        out_type=jax.ShapeDtypeStruct((NW, _L), jnp.float32),
        scratch_types=[
            pltpu.VMEM((IPW,), jnp.int32),
            pltpu.VMEM((IPW * R,), jnp.int32),
            pltpu.VMEM((CHUNK, D), jnp.float32),
            pltpu.VMEM((RPC, D), jnp.float32),
            pltpu.VMEM((RPC,), jnp.float32),
            pltpu.VMEM((_L,), jnp.float32),
            pltpu.SemaphoreType.DMA,
        ],
    )
    def sc_kernel(cw_hbm, idx2_hbm, w1_hbm, w2_hbm, out_hbm,
                  idxu_v, idx2_v, u_rows, vn_rows, dots_v, accv, sem):
        wid = lax.axis_index("s") * NC + lax.axis_index("c")
        base = wid * IPW
        pltpu.sync_copy(cw_hbm.at[pl.ds(base, IPW)], idxu_v)
        pltpu.sync_copy(idx2_hbm.at[pl.ds(base * R, IPW * R)], idx2_v)

        def chunk_body(c, acc):
            cu = pltpu.async_copy(
                w1_hbm.at[idxu_v.at[pl.ds(c * CHUNK, CHUNK)]], u_rows, sem)
            cvs = []
            for g in range(NG):
                cvs.append(pltpu.async_copy(
                    w2_hbm.at[idx2_v.at[pl.ds(c * RPC + g * GSPLIT, GSPLIT)]],
                    vn_rows.at[pl.ds(g * GSPLIT, GSPLIT)], sem))
            cu.wait()
            for cv in cvs:
                cv.wait()

            def item_body(i, _):
                u = [u_rows[i, pl.ds(q * _L, _L)] for q in range(DV)]

                def dot_body(j, _):
                    d = i * R + j
                    p = u[0] * vn_rows[d, pl.ds(0, _L)]
                    for q in range(1, DV):
                        p = p + u[q] * vn_rows[d, pl.ds(q * _L, _L)]
                    s = jnp.sum(p)
                    dots_v[d] = jnp.where(j == 0, s, -s)
                    return 0

                return lax.fori_loop(0, R, dot_body, 0)

            lax.fori_loop(0, CHUNK, item_body, 0)

            def grp_body(g, a):
                return a + _log_sigmoid_vec(dots_v[pl.ds(g * _L, _L)])

            return lax.fori_loop(0, GROUPS, grp_body, acc)

        acc = lax.fori_loop(0, NCHUNK, chunk_body, jnp.zeros((_L,), jnp.float32))
        accv[...] = acc
        pltpu.sync_copy(accv, out_hbm.at[wid])

    return sc_kernel


def kernel(context_words, targets, negative_samples, w1, w2):
    B = context_words.shape[0]
    K = negative_samples.shape[1]
    D = w1.shape[1]
    info = plsc.get_sparse_core_info()
    NC, NS = info.num_cores, info.num_subcores
    idx2 = jnp.concatenate([targets[:, None], negative_samples], axis=1).reshape(-1)
    partials = _make_sc_kernel(B, D, K, NC, NS)(context_words, idx2, w1, w2)
    return -jnp.sum(partials)


# speedup vs baseline: 4.9341x; 4.9341x over previous
import functools

import jax
import jax.numpy as jnp
from jax import lax
from jax.experimental import pallas as pl
from jax.experimental.pallas import tpu as pltpu
from jax.experimental.pallas import tpu_sc as plsc

_L = 16


def _log_sigmoid_vec(x):
    e = jnp.exp(-jnp.abs(x))
    t = e / (e + 2.0)
    w = t * t
    poly = 1.0 + w * (1.0 / 3.0 + w * (1.0 / 5.0 + w * (1.0 / 7.0 + w * (1.0 / 9.0))))
    log1p_e = 2.0 * t * poly
    return jnp.minimum(x, 0.0) - log1p_e


def _make_sc_kernel(B, D, K, NC, NS):
    NW = NC * NS
    R = K + 1
    IPW = B // NW
    CHUNK = 16
    NCHUNK = IPW // CHUNK
    RPC = CHUNK * R
    GSPLIT = 112
    NG = RPC // GSPLIT
    GROUPS = RPC // _L
    DV = D // _L

    mesh = plsc.VectorSubcoreMesh(core_axis_name="c", subcore_axis_name="s")

    @functools.partial(
        pl.kernel,
        mesh=mesh,
        compiler_params=pltpu.CompilerParams(
            needs_layout_passes=False, use_tc_tiling_on_sc=False),
        out_type=jax.ShapeDtypeStruct((NW, _L), jnp.float32),
        scratch_types=[
            pltpu.VMEM((IPW,), jnp.int32),
            pltpu.VMEM((IPW * R,), jnp.int32),
            pltpu.VMEM((CHUNK, D), jnp.float32),
            pltpu.VMEM((RPC, D), jnp.float32),
            pltpu.VMEM((RPC * (_L + 1),), jnp.float32),
            pltpu.VMEM((_L,), jnp.float32),
            pltpu.SemaphoreType.DMA,
        ],
    )
    def sc_kernel(cw_hbm, idx2_hbm, w1_hbm, w2_hbm, out_hbm,
                  idxu_v, idx2_v, u_rows, vn_rows, pbuf, accv, sem):
        wid = lax.axis_index("s") * NC + lax.axis_index("c")
        base = wid * IPW
        pltpu.sync_copy(cw_hbm.at[pl.ds(base, IPW)], idxu_v)
        pltpu.sync_copy(idx2_hbm.at[pl.ds(base * R, IPW * R)], idx2_v)

        def chunk_body(c, acc):
            cu = pltpu.async_copy(
                w1_hbm.at[idxu_v.at[pl.ds(c * CHUNK, CHUNK)]], u_rows, sem)
            cvs = []
            for g in range(NG):
                cvs.append(pltpu.async_copy(
                    w2_hbm.at[idx2_v.at[pl.ds(c * RPC + g * GSPLIT, GSPLIT)]],
                    vn_rows.at[pl.ds(g * GSPLIT, GSPLIT)], sem))
            cu.wait()
            for cv in cvs:
                cv.wait()

            def item_body(i, _):
                u = [u_rows[i, pl.ds(q * _L, _L)] for q in range(DV)]

                def dot_body(j, _):
                    d = i * R + j
                    p = u[0] * vn_rows[d, pl.ds(0, _L)]
                    for q in range(1, DV):
                        p = p + u[q] * vn_rows[d, pl.ds(q * _L, _L)]
                    pbuf[pl.ds(d * (_L + 1), _L)] = jnp.where(j == 0, p, -p)
                    return 0

                return lax.fori_loop(0, R, dot_body, 0)

            lax.fori_loop(0, CHUNK, item_body, 0)

            lane = lax.iota(jnp.int32, _L)

            def grp_body(g, a):
                addr = (g * _L + lane) * (_L + 1)
                s = plsc.load_gather(pbuf, [addr])
                for j in range(1, _L):
                    s = s + plsc.load_gather(pbuf, [addr + j])
                return a + _log_sigmoid_vec(s)

            return lax.fori_loop(0, GROUPS, grp_body, acc)

        acc = lax.fori_loop(0, NCHUNK, chunk_body, jnp.zeros((_L,), jnp.float32))
        accv[...] = acc
        pltpu.sync_copy(accv, out_hbm.at[wid])

    return sc_kernel


def kernel(context_words, targets, negative_samples, w1, w2):
    B = context_words.shape[0]
    K = negative_samples.shape[1]
    D = w1.shape[1]
    info = plsc.get_sparse_core_info()
    NC, NS = info.num_cores, info.num_subcores
    idx2 = jnp.concatenate([targets[:, None], negative_samples], axis=1).reshape(-1)
    partials = _make_sc_kernel(B, D, K, NC, NS)(context_words, idx2, w1, w2)
    return -jnp.sum(partials)


# speedup vs baseline: 5.1654x; 1.0469x over previous
import functools

import jax
import jax.numpy as jnp
from jax import lax
from jax.experimental import pallas as pl
from jax.experimental.pallas import tpu as pltpu
from jax.experimental.pallas import tpu_sc as plsc

_L = 16


def _log_sigmoid_vec(x):
    e = jnp.exp(-jnp.abs(x))
    t = e / (e + 2.0)
    w = t * t
    poly = 1.0 + w * (1.0 / 3.0 + w * (1.0 / 5.0 + w * (1.0 / 7.0 + w * (1.0 / 9.0))))
    log1p_e = 2.0 * t * poly
    return jnp.minimum(x, 0.0) - log1p_e


def _make_sc_kernel(B, D, K, NC, NS):
    NW = NC * NS
    R = K + 1
    IPW = B // NW
    CHUNK = 16
    NCHUNK = IPW // CHUNK
    RPC = CHUNK * R
    GSPLIT = 112
    NG = RPC // GSPLIT
    GROUPS = RPC // _L
    DV = D // _L

    mesh = plsc.VectorSubcoreMesh(core_axis_name="c", subcore_axis_name="s")

    @functools.partial(
        pl.kernel,
        mesh=mesh,
        compiler_params=pltpu.CompilerParams(
            needs_layout_passes=False, use_tc_tiling_on_sc=False),
        out_type=jax.ShapeDtypeStruct((NW, _L), jnp.float32),
        scratch_types=[
            pltpu.VMEM((IPW,), jnp.int32),
            pltpu.VMEM((IPW * R,), jnp.int32),
            pltpu.VMEM((CHUNK, D), jnp.float32),
            pltpu.VMEM((CHUNK, D), jnp.float32),
            pltpu.VMEM((RPC, D), jnp.float32),
            pltpu.VMEM((RPC, D), jnp.float32),
            pltpu.VMEM((RPC * (_L + 1),), jnp.float32),
            pltpu.VMEM((_L,), jnp.float32),
            pltpu.SemaphoreType.DMA,
            pltpu.SemaphoreType.DMA,
        ],
    )
    def sc_kernel(cw_hbm, idx2_hbm, w1_hbm, w2_hbm, out_hbm,
                  idxu_v, idx2_v, u0_rows, u1_rows, v0_rows, v1_rows,
                  pbuf, accv, sem0, sem1):
        wid = lax.axis_index("s") * NC + lax.axis_index("c")
        base = wid * IPW
        pltpu.sync_copy(cw_hbm.at[pl.ds(base, IPW)], idxu_v)
        pltpu.sync_copy(idx2_hbm.at[pl.ds(base * R, IPW * R)], idx2_v)

        def start_chunk(c, ub, vb, sem):
            pltpu.async_copy(
                w1_hbm.at[idxu_v.at[pl.ds(c * CHUNK, CHUNK)]], ub, sem)
            for g in range(NG):
                pltpu.async_copy(
                    w2_hbm.at[idx2_v.at[pl.ds(c * RPC + g * GSPLIT, GSPLIT)]],
                    vb.at[pl.ds(g * GSPLIT, GSPLIT)], sem)

        def wait_chunk(ub, vb, sem):
            pltpu.make_async_copy(
                w1_hbm.at[idxu_v.at[pl.ds(0, CHUNK)]], ub, sem).wait()
            for g in range(NG):
                pltpu.make_async_copy(
                    w2_hbm.at[idx2_v.at[pl.ds(0, GSPLIT)]],
                    vb.at[pl.ds(g * GSPLIT, GSPLIT)], sem).wait()

        lane = lax.iota(jnp.int32, _L)

        def compute_chunk(ub, vb, acc):
            def item_body(i, _):
                u = [ub[i, pl.ds(q * _L, _L)] for q in range(DV)]
                nu = [-uq for uq in u]
                for j in range(R):
                    d = i * R + j
                    c0 = u if j == 0 else nu
                    p = c0[0] * vb[d, pl.ds(0, _L)]
                    for q in range(1, DV):
                        p = p + c0[q] * vb[d, pl.ds(q * _L, _L)]
                    pbuf[pl.ds(d * (_L + 1), _L)] = p
                return 0

            lax.fori_loop(0, CHUNK, item_body, 0)

            def grp_body(g, a):
                addr = (g * _L + lane) * (_L + 1)
                s = plsc.load_gather(pbuf, [addr])
                for j in range(1, _L):
                    s = s + plsc.load_gather(pbuf, [addr + j])
                return a + _log_sigmoid_vec(s)

            return lax.fori_loop(0, GROUPS, grp_body, acc)

        start_chunk(0, u0_rows, v0_rows, sem0)

        def pair_body(h, acc):
            c = 2 * h
            start_chunk(c + 1, u1_rows, v1_rows, sem1)
            wait_chunk(u0_rows, v0_rows, sem0)
            acc = compute_chunk(u0_rows, v0_rows, acc)

            @pl.when(c + 2 < NCHUNK)
            def _():
                start_chunk(c + 2, u0_rows, v0_rows, sem0)

            wait_chunk(u1_rows, v1_rows, sem1)
            return compute_chunk(u1_rows, v1_rows, acc)

        acc = lax.fori_loop(0, NCHUNK // 2, pair_body,
                            jnp.zeros((_L,), jnp.float32))
        accv[...] = acc
        pltpu.sync_copy(accv, out_hbm.at[wid])

    return sc_kernel


def kernel(context_words, targets, negative_samples, w1, w2):
    B = context_words.shape[0]
    K = negative_samples.shape[1]
    D = w1.shape[1]
    info = plsc.get_sparse_core_info()
    NC, NS = info.num_cores, info.num_subcores
    idx2 = jnp.concatenate([targets[:, None], negative_samples], axis=1).reshape(-1)
    partials = _make_sc_kernel(B, D, K, NC, NS)(context_words, idx2, w1, w2)
    return -jnp.sum(partials)
